# Initial kernel scaffold; baseline (speedup 1.0000x reference)
import jax
import jax.numpy as jnp
from jax.experimental import pallas as pl


def kernel(x, base_W, base_b, lora_A, lora_B, router_W, router_b):
    raise NotImplementedError("write your pallas kernel here")


# speedup vs baseline: 5.1501x; 5.1501x over previous
import functools

import jax
import jax.numpy as jnp
from jax.experimental import pallas as pl
from jax.experimental.pallas import tpu as pltpu

B, S, D_IN, D_OUT = 4, 2048, 1024, 1024
E, R, TOP_K = 8, 8, 2
SCALING = 32 / R

TOKENS = B * S
BLOCK_T = 256


def _fused_kernel(x_ref, w_ref, b_ref, acat_ref, bcat_ref, rw_ref, rb_ref,
                  expand_ref, out_ref):
    xb = x_ref[...]

    base = jax.lax.dot_general(
        xb, w_ref[...], (((1,), (1,)), ((), ())),
        preferred_element_type=jnp.float32)

    logits = jax.lax.dot_general(
        xb, rw_ref[...], (((1,), (1,)), ((), ())),
        preferred_element_type=jnp.float32) + rb_ref[...]

    m1 = jnp.max(logits, axis=-1, keepdims=True)
    lane = jax.lax.broadcasted_iota(jnp.int32, logits.shape, 1)
    is_max = logits == m1
    first_max = jnp.min(jnp.where(is_max, lane, E), axis=-1, keepdims=True)
    wo_top1 = jnp.where(lane == first_max, -jnp.inf, logits)
    kth = jnp.max(wo_top1, axis=-1, keepdims=True)
    masked = jnp.where(logits < kth, -1e9, logits)

    e = jnp.exp(masked - m1)
    gates = e / jnp.sum(e, axis=-1, keepdims=True)

    gates64 = jax.lax.dot_general(
        gates, expand_ref[...], (((1,), (0,)), ((), ())),
        preferred_element_type=jnp.float32)

    ax = jax.lax.dot_general(
        xb, acat_ref[...], (((1,), (0,)), ((), ())),
        preferred_element_type=jnp.float32)
    lora = jax.lax.dot_general(
        ax * gates64, bcat_ref[...], (((1,), (0,)), ((), ())),
        preferred_element_type=jnp.float32)

    out_ref[...] = base + b_ref[...] + SCALING * lora


@jax.jit
def kernel(x, base_W, base_b, lora_A, lora_B, router_W, router_b):
    x2 = x.reshape(TOKENS, D_IN)
    a_cat = jnp.transpose(lora_A, (1, 0, 2)).reshape(D_IN, E * R)
    b_cat = lora_B.reshape(E * R, D_OUT)
    bias = base_b.reshape(1, D_OUT)
    rb = router_b.reshape(1, E)
    expand = (jax.lax.broadcasted_iota(jnp.int32, (E, E * R), 0)
              == jax.lax.broadcasted_iota(jnp.int32, (E, E * R), 1) // R
              ).astype(jnp.float32)

    grid = (TOKENS // BLOCK_T,)
    out = pl.pallas_call(
        _fused_kernel,
        grid=grid,
        in_specs=[
            pl.BlockSpec((BLOCK_T, D_IN), lambda i: (i, 0)),
            pl.BlockSpec((D_OUT, D_IN), lambda i: (0, 0)),
            pl.BlockSpec((1, D_OUT), lambda i: (0, 0)),
            pl.BlockSpec((D_IN, E * R), lambda i: (0, 0)),
            pl.BlockSpec((E * R, D_OUT), lambda i: (0, 0)),
            pl.BlockSpec((E, D_IN), lambda i: (0, 0)),
            pl.BlockSpec((1, E), lambda i: (0, 0)),
            pl.BlockSpec((E, E * R), lambda i: (0, 0)),
        ],
        out_specs=pl.BlockSpec((BLOCK_T, D_OUT), lambda i: (i, 0)),
        out_shape=jax.ShapeDtypeStruct((TOKENS, D_OUT), jnp.float32),
        compiler_params=pltpu.CompilerParams(
            dimension_semantics=("arbitrary",)),
    )(x2, base_W, bias, a_cat, b_cat, router_W, rb, expand)
    return out.reshape(B, S, D_OUT)
